# Initial kernel scaffold; baseline (speedup 1.0000x reference)
import jax
import jax.numpy as jnp
from jax.experimental import pallas as pl


def kernel(info_state, rank_embs, suit_embs, card_embs):
    raise NotImplementedError("write your pallas kernel here")


# speedup vs baseline: 8.1198x; 8.1198x over previous
import jax
import jax.numpy as jnp
from jax import lax
from jax.experimental import pallas as pl
from jax.experimental.pallas import tpu as pltpu
from jax.experimental.pallas import tpu_sc as plsc

BATCH = 16384
FEAT = 64
DIM = 64
NUM_HOLE = 2
TOTAL_BOARD = 5
NCARDS = NUM_HOLE + TOTAL_BOARD
N_COMB = 104
TAIL = 16
CARD0 = TAIL - NCARDS

NC = 2
NS = 16
NW = NC * NS
BPW = BATCH // NW
GROUPS = BPW // 16

_MESH = plsc.VectorSubcoreMesh(core_axis_name="c", subcore_axis_name="s")
_PARAMS = pltpu.CompilerParams(use_tc_tiling_on_sc=False,
                               needs_layout_passes=False)


def _wid():
    return lax.axis_index("c") * NS + lax.axis_index("s")


def _build_comb_body(rank_hbm, suit_hbm, card_hbm, comb_hbm,
                     rank_v, suit_v, card_v, blk):
    wid = _wid()

    @pl.when(wid < N_COMB // 8)
    def _():
        pltpu.sync_copy(rank_hbm, rank_v)
        pltpu.sync_copy(suit_hbm, suit_v)
        pltpu.sync_copy(card_hbm, card_v)
        for i in range(8):
            row = wid * 8 + i
            g = jnp.where(row < 52, 0, 1)
            cc = row - g * 52
            cbase = (g * 52 + cc) * DIM
            rbase = (g * 13 + cc // 4) * DIM
            sbase = (g * 4 + cc % 4) * DIM
            for q in range(DIM // 16):
                o = q * 16
                blk[pl.ds(i * DIM + o, 16)] = (card_v[pl.ds(cbase + o, 16)]
                                               + rank_v[pl.ds(rbase + o, 16)]
                                               + suit_v[pl.ds(sbase + o, 16)])
        pltpu.sync_copy(blk, comb_hbm.at[pl.ds(wid * 8 * DIM, 8 * DIM)])


def _lookup_body(info_hbm, comb_hbm, out_hbm,
                 info16, comb_v, addr_refs, stage):
    wid = _wid()
    base = wid * BPW
    pltpu.sync_copy(comb_hbm, comb_v)
    pltpu.sync_copy(info_hbm.at[pl.ds(base, BPW), pl.ds(FEAT - TAIL, TAIL)],
                    info16)
    lanes = lax.iota(jnp.int32, 16)

    for g in range(GROUPS):
        rows = g * 16 + lanes
        for j in range(NCARDS):
            col = jnp.full((16,), CARD0 + j, jnp.int32)
            c = plsc.load_gather(info16, [rows, col]).astype(jnp.int32)
            off = 52 if j < TOTAL_BOARD else 0
            addr_refs[j][pl.ds(g * 16, 16)] = (c + off) * DIM

    def group_body(g, _):
        sl = pl.ds(g * 16, 16)
        rows = g * 16 + lanes
        a = [addr_refs[j][sl] for j in range(NCARDS)]
        for d in range(DIM):
            dh = jnp.full((16,), d, jnp.int32)
            db = jnp.full((16,), DIM + d, jnp.int32)
            hole = (plsc.load_gather(comb_v, [a[5] + d])
                    + plsc.load_gather(comb_v, [a[6] + d]))
            plsc.store_scatter(stage, [rows, dh], hole)
            board = plsc.load_gather(comb_v, [a[0] + d])
            for j in range(1, TOTAL_BOARD):
                board = board + plsc.load_gather(comb_v, [a[j] + d])
            plsc.store_scatter(stage, [rows, db], board)
        return _

    lax.fori_loop(0, GROUPS, group_body, None)
    pltpu.sync_copy(stage, out_hbm.at[pl.ds(base, BPW)])


def kernel(info_state, rank_embs, suit_embs, card_embs):
    comb = pl.kernel(
        _build_comb_body,
        out_type=jax.ShapeDtypeStruct((N_COMB * DIM,), jnp.float32),
        mesh=_MESH,
        compiler_params=_PARAMS,
        scratch_types=[
            pltpu.VMEM((2 * 13 * DIM,), jnp.float32),
            pltpu.VMEM((2 * 4 * DIM,), jnp.float32),
            pltpu.VMEM((2 * 52 * DIM,), jnp.float32),
            pltpu.VMEM((8 * DIM,), jnp.float32),
        ],
    )(rank_embs.reshape(-1), suit_embs.reshape(-1), card_embs.reshape(-1))

    out = pl.kernel(
        _lookup_body,
        out_type=jax.ShapeDtypeStruct((BATCH, 2 * DIM), jnp.float32),
        mesh=_MESH,
        compiler_params=_PARAMS,
        scratch_types=[
            pltpu.VMEM((BPW, TAIL), jnp.float32),
            pltpu.VMEM((N_COMB * DIM,), jnp.float32),
            [pltpu.VMEM((BPW,), jnp.int32) for _ in range(NCARDS)],
            pltpu.VMEM((BPW, 2 * DIM), jnp.float32),
        ],
    )(info_state, comb)
    return out
